# baseline (speedup 1.0000x reference)
import jax
import jax.numpy as jnp
from jax.experimental import pallas as pl
from jax.experimental.pallas import tpu as pltpu

_VMEM_LIMIT_BYTES = 48 * 1024 * 1024


def _make_kernel(C, W, Ho, Wo):
    HW = 2 * Ho * W
    half = HW // 2
    N2 = Ho * W

    def body(x_ref, w_ref, b_ref, o_ref, patch_ref, acc_ref):
        for ky in range(3):
            for kx in range(3):
                tap = ky * 3 + kx
                cp = (kx + 1) % 2
                d = -1 if kx == 0 else 0
                s = cp * half + (W // 2) * (ky - 1) + d
                n_lo = Wo if ky == 0 else 0
                n_hi = min(N2, HW - s)
                v = x_ref[0, :, n_lo + s:n_hi + s]
                if kx == 0:
                    idx = jax.lax.broadcasted_iota(
                        jnp.int32, (C, n_hi - n_lo), 1) + n_lo
                    v = jnp.where((idx & (W - 1)) == 0, jnp.bfloat16(0), v)
                patch_ref[tap * C:(tap + 1) * C, n_lo:n_hi] = v
                if n_lo:
                    patch_ref[tap * C:(tap + 1) * C, 0:n_lo] = jnp.zeros(
                        (C, n_lo), jnp.bfloat16)

        acc_ref[...] = jnp.dot(w_ref[...], patch_ref[...],
                               preferred_element_type=jnp.float32) + b_ref[...]
        for oy in range(Ho):
            o_ref[0, :, oy * Wo:(oy + 1) * Wo] = acc_ref[:, oy * W:oy * W + Wo]

    return body


def kernel(x, conv_w, conv_b):
    B, C, H, W = x.shape
    OC = conv_w.shape[0]
    Ho, Wo = H // 2, W // 2
    N2 = Ho * W

    xc = jnp.concatenate([x[:, :, :, 0::2], x[:, :, :, 1::2]], axis=2)
    xc = xc.astype(jnp.bfloat16).reshape(B, C, H * W)

    w2 = conv_w.transpose(0, 2, 3, 1).reshape(OC, 9 * C).astype(jnp.bfloat16)
    b2 = conv_b.reshape(OC, 1).astype(jnp.float32)

    out = pl.pallas_call(
        _make_kernel(C, W, Ho, Wo),
        out_shape=jax.ShapeDtypeStruct((B, OC, Ho * Wo), jnp.float32),
        grid=(B,),
        in_specs=[
            pl.BlockSpec((1, C, H * W), lambda i: (i, 0, 0)),
            pl.BlockSpec((OC, 9 * C), lambda i: (0, 0)),
            pl.BlockSpec((OC, 1), lambda i: (0, 0)),
        ],
        out_specs=pl.BlockSpec((1, OC, Ho * Wo), lambda i: (i, 0, 0)),
        scratch_shapes=[pltpu.VMEM((9 * C, N2), jnp.bfloat16),
                        pltpu.VMEM((OC, N2), jnp.float32)],
        compiler_params=pltpu.CompilerParams(
            dimension_semantics=("parallel",),
            vmem_limit_bytes=_VMEM_LIMIT_BYTES),
    )(xc, w2, b2)
    return out.reshape(B, OC, Ho, Wo)


# speedup vs baseline: 2.1502x; 2.1502x over previous
import jax
import jax.numpy as jnp
from jax.experimental import pallas as pl
from jax.experimental.pallas import tpu as pltpu

_VMEM_LIMIT_BYTES = 48 * 1024 * 1024


def _make_kernel(C, Ho, Wo):
    N = Ho * Wo

    def body(x_ref, w_ref, b_ref, o_ref, patch_ref):
        for ky in range(3):
            for kx in range(3):
                tap = ky * 3 + kx
                rp, ady = (1, -1) if ky == 0 else (ky - 1, 0)
                wp, adx = (1, -1) if kx == 0 else (kx - 1, 0)
                zp = rp * 2 + wp
                s = Wo * ady + adx
                n_lo = max(Wo if ky == 0 else 0, -s if s < 0 else 0)
                v = x_ref[0, zp * C:(zp + 1) * C, n_lo + s:N + s]
                if kx == 0:
                    idx = jax.lax.broadcasted_iota(
                        jnp.int32, (C, N - n_lo), 1) + n_lo
                    v = jnp.where((idx & (Wo - 1)) == 0, jnp.bfloat16(0), v)
                patch_ref[tap * C:(tap + 1) * C, n_lo:N] = v
                if n_lo:
                    patch_ref[tap * C:(tap + 1) * C, 0:n_lo] = jnp.zeros(
                        (C, n_lo), jnp.bfloat16)

        o_ref[0] = (jnp.dot(w_ref[...], patch_ref[...],
                            preferred_element_type=jnp.float32)
                    + b_ref[...])

    return body


def kernel(x, conv_w, conv_b):
    B, C, H, W = x.shape
    OC = conv_w.shape[0]
    Ho, Wo = H // 2, W // 2
    N = Ho * Wo

    xps = x.reshape(B, C, Ho, 2, Wo, 2).transpose(0, 3, 5, 1, 2, 4)
    xps = xps.astype(jnp.bfloat16).reshape(B, 4 * C, N)

    w2 = conv_w.transpose(0, 2, 3, 1).reshape(OC, 9 * C).astype(jnp.bfloat16)
    b2 = conv_b.reshape(OC, 1).astype(jnp.float32)

    out = pl.pallas_call(
        _make_kernel(C, Ho, Wo),
        out_shape=jax.ShapeDtypeStruct((B, OC, N), jnp.float32),
        grid=(B,),
        in_specs=[
            pl.BlockSpec((1, 4 * C, N), lambda i: (i, 0, 0)),
            pl.BlockSpec((OC, 9 * C), lambda i: (0, 0)),
            pl.BlockSpec((OC, 1), lambda i: (0, 0)),
        ],
        out_specs=pl.BlockSpec((1, OC, N), lambda i: (i, 0, 0)),
        scratch_shapes=[pltpu.VMEM((9 * C, N), jnp.bfloat16)],
        compiler_params=pltpu.CompilerParams(
            dimension_semantics=("parallel",),
            vmem_limit_bytes=_VMEM_LIMIT_BYTES),
    )(xps, w2, b2)
    return out.reshape(B, OC, Ho, Wo)
